# Initial kernel scaffold; baseline (speedup 1.0000x reference)
import jax
import jax.numpy as jnp
from jax.experimental import pallas as pl


def kernel(pos_f, vel_f, edge_index, W_pos, b_pos, W_vel, b_vel, W_node, b_node, W_self, W_nbr, b_proc, W_posdec, b_posdec, W_veldec, b_veldec):
    raise NotImplementedError("write your pallas kernel here")


# speedup vs baseline: 3.5831x; 3.5831x over previous
import functools

import jax
import jax.numpy as jnp
from jax import lax
from jax.experimental import pallas as pl
from jax.experimental.pallas import tpu as pltpu
from jax.experimental.pallas import tpu_sc as plsc

N = 50000
E = 800000
HID = 64
HHALF = 32

NTILES = 16
K = 128
KH = 64
ROWS_PT = 391
EPAD = NTILES * ROWS_PT * K
ICH = 8
NICH = ROWS_PT // ICH
ITAIL = ROWS_PT - NICH * ICH
QROWS = 12544
ZQ = QROWS // NTILES

R_TC = 2000


def _place4(hc, zero32):
    slabs = []
    for p in range(4):
        cols = [zero32] * p + [hc] + [zero32] * (3 - p)
        slabs.append(jnp.concatenate(cols, axis=1))
    return jnp.stack(slabs, axis=1).reshape(4 * hc.shape[0], 128)


def _encoder_body(pos_ref, vel_ref, wp_ref, bp_ref, wv_ref, bv_ref,
                  wn_ref, bn_ref, h_ref, slo_ref, shi_ref):
    pos_hid = jnp.maximum(
        jnp.dot(pos_ref[...], wp_ref[...], preferred_element_type=jnp.float32)
        + bp_ref[...], 0.0)
    vel_hid = jnp.maximum(
        jnp.dot(vel_ref[...], wv_ref[...], preferred_element_type=jnp.float32)
        + bv_ref[...], 0.0)
    cat = jnp.concatenate([pos_hid, vel_hid], axis=1)
    h = jnp.maximum(
        jnp.dot(cat, wn_ref[...], preferred_element_type=jnp.float32)
        + bn_ref[...], 0.0)
    h_ref[...] = h
    zero32 = jnp.zeros((h.shape[0], HHALF), jnp.float32)
    slo_ref[...] = _place4(h[:, :HHALF], zero32)
    shi_ref[...] = _place4(h[:, HHALF:], zero32)


def _encode(pos_f, vel_f, W_pos, b_pos, W_vel, b_vel, W_node, b_node):
    grid = (N // R_TC,)
    full = lambda shape: pl.BlockSpec(shape, lambda i: (0, 0))
    return pl.pallas_call(
        _encoder_body,
        grid=grid,
        in_specs=[
            pl.BlockSpec((R_TC, 2), lambda i: (i, 0)),
            pl.BlockSpec((R_TC, 2), lambda i: (i, 0)),
            full((2, HID)), full((1, HID)),
            full((2, HID)), full((1, HID)),
            full((2 * HID, HID)), full((1, HID)),
        ],
        out_specs=[
            pl.BlockSpec((R_TC, HID), lambda i: (i, 0)),
            pl.BlockSpec((4 * R_TC, 128), lambda i: (i, 0)),
            pl.BlockSpec((4 * R_TC, 128), lambda i: (i, 0)),
        ],
        out_shape=[
            jax.ShapeDtypeStruct((N, HID), jnp.float32),
            jax.ShapeDtypeStruct((4 * N, 128), jnp.float32),
            jax.ShapeDtypeStruct((4 * N, 128), jnp.float32),
        ],
    )(pos_f, vel_f, W_pos, b_pos, W_vel, b_vel, W_node, b_node)


def _sc_aggregate(scr_lo, scr_hi, src3, dst3, zq):
    mesh = plsc.VectorSubcoreMesh(core_axis_name="c", subcore_axis_name="s")

    @functools.partial(
        pl.kernel,
        mesh=mesh,
        out_type=jax.ShapeDtypeStruct((2, QROWS, 128), jnp.float32),
        scratch_types=[
            pltpu.VMEM((ICH, K), jnp.int32),
            pltpu.VMEM((ICH, K), jnp.int32),
            pltpu.VMEM((KH,), jnp.int32),
            pltpu.VMEM((KH,), jnp.int32),
            pltpu.VMEM((KH,), jnp.int32),
            pltpu.VMEM((KH,), jnp.int32),
            pltpu.VMEM((KH, 128), jnp.float32),
            pltpu.VMEM((KH, 128), jnp.float32),
            pltpu.VMEM_SHARED((QROWS, 128), jnp.float32),
            pltpu.SemaphoreType.DMA,
            pltpu.SemaphoreType.DMA,
        ],
    )
    def run(slo_hbm, shi_hbm, src_hbm, dst_hbm, zq_hbm, out_hbm,
            src_v, dst_v, ig0, ig1, q0, q1, rowsA, rowsB, acc, semA, semB):
        c = lax.axis_index("c")
        s = lax.axis_index("s")

        pltpu.sync_copy(zq_hbm, acc.at[pl.ds(s * ZQ, ZQ)])
        plsc.subcore_barrier()

        def do_row(tab, r):
            for g in range(8):
                sv = src_v[r, pl.ds(16 * g, 16)]
                dv = dst_v[r, pl.ds(16 * g, 16)]
                igv = sv * 4 + (dv & 3)
                qv = lax.shift_right_logical(dv, 2)
                if g < 4:
                    ig0[pl.ds(16 * g, 16)] = igv
                    q0[pl.ds(16 * g, 16)] = qv
                else:
                    ig1[pl.ds(16 * (g - 4), 16)] = igv
                    q1[pl.ds(16 * (g - 4), 16)] = qv
            cpA = pltpu.async_copy(tab.at[ig0], rowsA, semA)
            cpB = pltpu.async_copy(tab.at[ig1], rowsB, semB)
            cpA.wait()
            pltpu.sync_copy(rowsA, acc.at[q0], add=True)
            cpB.wait()
            pltpu.sync_copy(rowsB, acc.at[q1], add=True)

        def edge_loop(tab):
            def super_body(t, carry):
                off = pl.multiple_of(t * ICH, 8)
                pltpu.sync_copy(src_hbm.at[s, pl.ds(off, ICH)], src_v)
                pltpu.sync_copy(dst_hbm.at[s, pl.ds(off, ICH)], dst_v)
                for r in range(ICH):
                    do_row(tab, r)
                return carry

            lax.fori_loop(0, NICH, super_body, 0)
            pltpu.sync_copy(src_hbm.at[s, pl.ds(NICH * ICH, ITAIL)],
                            src_v.at[pl.ds(0, ITAIL)])
            pltpu.sync_copy(dst_hbm.at[s, pl.ds(NICH * ICH, ITAIL)],
                            dst_v.at[pl.ds(0, ITAIL)])
            for r in range(ITAIL):
                do_row(tab, r)

        @pl.when(c == 0)
        def _():
            edge_loop(slo_hbm)

        @pl.when(c == 1)
        def _():
            edge_loop(shi_hbm)

        plsc.subcore_barrier()

        pltpu.sync_copy(acc.at[pl.ds(s * ZQ, ZQ)],
                        out_hbm.at[c, pl.ds(s * ZQ, ZQ)])

    return run(scr_lo, scr_hi, src3, dst3, zq)


def _proc_body(h_ref, alo_ref, ahi_ref, ws_ref, wn_ref, bp_ref,
               wpd_ref, bpd_ref, wvd_ref, bvd_ref, pos_ref, vel_ref):
    agg = jnp.concatenate([alo_ref[...], ahi_ref[...]], axis=1)
    h2 = jnp.maximum(
        jnp.dot(h_ref[...], ws_ref[...], preferred_element_type=jnp.float32)
        + jnp.dot(agg, wn_ref[...], preferred_element_type=jnp.float32)
        + bp_ref[...], 0.0)
    pos_ref[...] = (
        jnp.dot(h2, wpd_ref[...], preferred_element_type=jnp.float32)
        + bpd_ref[...])
    vel_ref[...] = (
        jnp.dot(h2, wvd_ref[...], preferred_element_type=jnp.float32)
        + bvd_ref[...])


def _process(h, agg_lo, agg_hi, W_self, W_nbr, b_proc,
             W_posdec, b_posdec, W_veldec, b_veldec):
    grid = (N // R_TC,)
    full = lambda shape: pl.BlockSpec(shape, lambda i: (0, 0))
    return pl.pallas_call(
        _proc_body,
        grid=grid,
        in_specs=[
            pl.BlockSpec((R_TC, HID), lambda i: (i, 0)),
            pl.BlockSpec((R_TC, HHALF), lambda i: (i, 0)),
            pl.BlockSpec((R_TC, HHALF), lambda i: (i, 0)),
            full((HID, HID)), full((HID, HID)), full((1, HID)),
            full((HID, 2)), full((1, 2)),
            full((HID, 2)), full((1, 2)),
        ],
        out_specs=[
            pl.BlockSpec((R_TC, 2), lambda i: (i, 0)),
            pl.BlockSpec((R_TC, 2), lambda i: (i, 0)),
        ],
        out_shape=[
            jax.ShapeDtypeStruct((N, 2), jnp.float32),
            jax.ShapeDtypeStruct((N, 2), jnp.float32),
        ],
    )(h, agg_lo, agg_hi, W_self, W_nbr, b_proc,
      W_posdec, b_posdec, W_veldec, b_veldec)


def kernel(pos_f, vel_f, edge_index, W_pos, b_pos, W_vel, b_vel,
           W_node, b_node, W_self, W_nbr, b_proc,
           W_posdec, b_posdec, W_veldec, b_veldec):
    ei = edge_index.astype(jnp.int32)
    npad = EPAD - E
    pad_src = (jnp.arange(npad, dtype=jnp.int32) * 997) % N
    pad_dst = N + (jnp.arange(npad, dtype=jnp.int32) % 176)
    src3 = jnp.concatenate([ei[0], pad_src]).reshape(NTILES, ROWS_PT, K)
    dst3 = jnp.concatenate([ei[1], pad_dst]).reshape(NTILES, ROWS_PT, K)
    zq = jnp.zeros((ZQ, 128), jnp.float32)

    h, scr_lo, scr_hi = _encode(
        pos_f, vel_f, W_pos, b_pos.reshape(1, HID),
        W_vel, b_vel.reshape(1, HID), W_node, b_node.reshape(1, HID))
    acc2 = _sc_aggregate(scr_lo, scr_hi, src3, dst3, zq)
    agg_lo = acc2[0].reshape(4 * QROWS, HHALF)[:N]
    agg_hi = acc2[1].reshape(4 * QROWS, HHALF)[:N]
    pos_hat, vel_hat = _process(
        h, agg_lo, agg_hi, W_self, W_nbr, b_proc.reshape(1, HID),
        W_posdec, b_posdec.reshape(1, 2), W_veldec, b_veldec.reshape(1, 2))
    return (pos_hat, vel_hat)


# speedup vs baseline: 4.8752x; 1.3606x over previous
import functools

import jax
import jax.numpy as jnp
from jax import lax
from jax.experimental import pallas as pl
from jax.experimental.pallas import tpu as pltpu
from jax.experimental.pallas import tpu_sc as plsc

N = 50000
E = 800000
HID = 64
HHALF = 32

NTILES = 16
K = 128
KQ = 32
ROWS_PT = 391
EPAD = NTILES * ROWS_PT * K
ICH = 8
NICH = ROWS_PT // ICH
ITAIL = ROWS_PT - NICH * ICH
QROWS = 12544
ZQ = QROWS // NTILES

R_TC = 2000


def _place4(hc, zero32):
    slabs = []
    for p in range(4):
        cols = [zero32] * p + [hc] + [zero32] * (3 - p)
        slabs.append(jnp.concatenate(cols, axis=1))
    return jnp.stack(slabs, axis=1).reshape(4 * hc.shape[0], 128)


def _encoder_body(pos_ref, vel_ref, wp_ref, bp_ref, wv_ref, bv_ref,
                  wn_ref, bn_ref, h_ref, slo_ref, shi_ref):
    pos_hid = jnp.maximum(
        jnp.dot(pos_ref[...], wp_ref[...], preferred_element_type=jnp.float32)
        + bp_ref[...], 0.0)
    vel_hid = jnp.maximum(
        jnp.dot(vel_ref[...], wv_ref[...], preferred_element_type=jnp.float32)
        + bv_ref[...], 0.0)
    cat = jnp.concatenate([pos_hid, vel_hid], axis=1)
    h = jnp.maximum(
        jnp.dot(cat, wn_ref[...], preferred_element_type=jnp.float32)
        + bn_ref[...], 0.0)
    h_ref[...] = h
    zero32 = jnp.zeros((h.shape[0], HHALF), jnp.float32)
    slo_ref[...] = _place4(h[:, :HHALF], zero32)
    shi_ref[...] = _place4(h[:, HHALF:], zero32)


def _encode(pos_f, vel_f, W_pos, b_pos, W_vel, b_vel, W_node, b_node):
    grid = (N // R_TC,)
    full = lambda shape: pl.BlockSpec(shape, lambda i: (0, 0))
    return pl.pallas_call(
        _encoder_body,
        grid=grid,
        in_specs=[
            pl.BlockSpec((R_TC, 2), lambda i: (i, 0)),
            pl.BlockSpec((R_TC, 2), lambda i: (i, 0)),
            full((2, HID)), full((1, HID)),
            full((2, HID)), full((1, HID)),
            full((2 * HID, HID)), full((1, HID)),
        ],
        out_specs=[
            pl.BlockSpec((R_TC, HID), lambda i: (i, 0)),
            pl.BlockSpec((4 * R_TC, 128), lambda i: (i, 0)),
            pl.BlockSpec((4 * R_TC, 128), lambda i: (i, 0)),
        ],
        out_shape=[
            jax.ShapeDtypeStruct((N, HID), jnp.float32),
            jax.ShapeDtypeStruct((4 * N, 128), jnp.float32),
            jax.ShapeDtypeStruct((4 * N, 128), jnp.float32),
        ],
    )(pos_f, vel_f, W_pos, b_pos, W_vel, b_vel, W_node, b_node)


def _sc_aggregate(scr_lo, scr_hi, src3, dst3, zq):
    mesh = plsc.VectorSubcoreMesh(core_axis_name="c", subcore_axis_name="s")

    @functools.partial(
        pl.kernel,
        mesh=mesh,
        out_type=jax.ShapeDtypeStruct((2, QROWS, 128), jnp.float32),
        scratch_types=[
            pltpu.VMEM((ICH, K), jnp.int32),
            pltpu.VMEM((ICH, K), jnp.int32),
            [pltpu.VMEM((KQ,), jnp.int32) for _ in range(4)],
            [pltpu.VMEM((KQ,), jnp.int32) for _ in range(4)],
            [pltpu.VMEM((KQ, 128), jnp.float32) for _ in range(4)],
            pltpu.VMEM_SHARED((QROWS, 128), jnp.float32),
            [pltpu.SemaphoreType.DMA for _ in range(4)],
            [pltpu.SemaphoreType.DMA for _ in range(4)],
        ],
    )
    def run(slo_hbm, shi_hbm, src_hbm, dst_hbm, zq_hbm, out_hbm,
            src_v, dst_v, ig, q, rows, acc, semG, semS):
        c = lax.axis_index("c")
        s = lax.axis_index("s")

        pltpu.sync_copy(zq_hbm, acc.at[pl.ds(s * ZQ, ZQ)])
        plsc.subcore_barrier()

        def waitS(b):
            pltpu.make_async_copy(rows[b], acc.at[q[b]], semS[b]).wait()

        def unit(tab, ul, prior):
            b = ul % 4
            r = ul // 4
            qq = ul % 4

            if ul >= 4 or prior is True:
                waitS(b)
            elif prior is not None:
                pl.when(prior)(lambda: waitS(b))

            for g in (2 * qq, 2 * qq + 1):
                sv = src_v[r, pl.ds(16 * g, 16)]
                dv = dst_v[r, pl.ds(16 * g, 16)]
                ig[b][pl.ds(16 * (g % 2), 16)] = sv * 4 + (dv & 3)
                q[b][pl.ds(16 * (g % 2), 16)] = lax.shift_right_logical(dv, 2)
            pltpu.async_copy(tab.at[ig[b]], rows[b], semG[b])

            us = ul - 3
            bs = (ul + 1) % 4

            def fire(bs=bs):
                pltpu.make_async_copy(tab.at[ig[bs]], rows[bs],
                                      semG[bs]).wait()
                pltpu.async_copy(rows[bs], acc.at[q[bs]], semS[bs], add=True)

            if us >= 0 or prior is True:
                fire()
            elif prior is not None:
                pl.when(prior)(fire)

        def edge_loop(tab):
            def super_body(t, carry):
                off = pl.multiple_of(t * ICH, 8)
                pltpu.sync_copy(src_hbm.at[s, pl.ds(off, ICH)], src_v)
                pltpu.sync_copy(dst_hbm.at[s, pl.ds(off, ICH)], dst_v)
                for ul in range(4 * ICH):
                    unit(tab, ul, t > 0)
                return carry

            lax.fori_loop(0, NICH, super_body, 0)
            pltpu.sync_copy(src_hbm.at[s, pl.ds(NICH * ICH, ITAIL)],
                            src_v.at[pl.ds(0, ITAIL)])
            pltpu.sync_copy(dst_hbm.at[s, pl.ds(NICH * ICH, ITAIL)],
                            dst_v.at[pl.ds(0, ITAIL)])
            for ul in range(4 * ITAIL):
                unit(tab, ul, True)
            for us in range(4 * ITAIL - 3, 4 * ITAIL):
                bs = us % 4
                pltpu.make_async_copy(tab.at[ig[bs]], rows[bs],
                                      semG[bs]).wait()
                pltpu.async_copy(rows[bs], acc.at[q[bs]], semS[bs], add=True)
            for b in range(4):
                waitS(b)

        @pl.when(c == 0)
        def _():
            edge_loop(slo_hbm)

        @pl.when(c == 1)
        def _():
            edge_loop(shi_hbm)

        plsc.subcore_barrier()

        pltpu.sync_copy(acc.at[pl.ds(s * ZQ, ZQ)],
                        out_hbm.at[c, pl.ds(s * ZQ, ZQ)])

    return run(scr_lo, scr_hi, src3, dst3, zq)


def _proc_body(h_ref, alo_ref, ahi_ref, ws_ref, wn_ref, bp_ref,
               wpd_ref, bpd_ref, wvd_ref, bvd_ref, pos_ref, vel_ref):
    agg = jnp.concatenate([alo_ref[...], ahi_ref[...]], axis=1)
    h2 = jnp.maximum(
        jnp.dot(h_ref[...], ws_ref[...], preferred_element_type=jnp.float32)
        + jnp.dot(agg, wn_ref[...], preferred_element_type=jnp.float32)
        + bp_ref[...], 0.0)
    pos_ref[...] = (
        jnp.dot(h2, wpd_ref[...], preferred_element_type=jnp.float32)
        + bpd_ref[...])
    vel_ref[...] = (
        jnp.dot(h2, wvd_ref[...], preferred_element_type=jnp.float32)
        + bvd_ref[...])


def _process(h, agg_lo, agg_hi, W_self, W_nbr, b_proc,
             W_posdec, b_posdec, W_veldec, b_veldec):
    grid = (N // R_TC,)
    full = lambda shape: pl.BlockSpec(shape, lambda i: (0, 0))
    return pl.pallas_call(
        _proc_body,
        grid=grid,
        in_specs=[
            pl.BlockSpec((R_TC, HID), lambda i: (i, 0)),
            pl.BlockSpec((R_TC, HHALF), lambda i: (i, 0)),
            pl.BlockSpec((R_TC, HHALF), lambda i: (i, 0)),
            full((HID, HID)), full((HID, HID)), full((1, HID)),
            full((HID, 2)), full((1, 2)),
            full((HID, 2)), full((1, 2)),
        ],
        out_specs=[
            pl.BlockSpec((R_TC, 2), lambda i: (i, 0)),
            pl.BlockSpec((R_TC, 2), lambda i: (i, 0)),
        ],
        out_shape=[
            jax.ShapeDtypeStruct((N, 2), jnp.float32),
            jax.ShapeDtypeStruct((N, 2), jnp.float32),
        ],
    )(h, agg_lo, agg_hi, W_self, W_nbr, b_proc,
      W_posdec, b_posdec, W_veldec, b_veldec)


def kernel(pos_f, vel_f, edge_index, W_pos, b_pos, W_vel, b_vel,
           W_node, b_node, W_self, W_nbr, b_proc,
           W_posdec, b_posdec, W_veldec, b_veldec):
    ei = edge_index.astype(jnp.int32)
    npad = EPAD - E
    pad_src = (jnp.arange(npad, dtype=jnp.int32) * 997) % N
    pad_dst = N + (jnp.arange(npad, dtype=jnp.int32) % 176)
    src3 = jnp.concatenate([ei[0], pad_src]).reshape(NTILES, ROWS_PT, K)
    dst3 = jnp.concatenate([ei[1], pad_dst]).reshape(NTILES, ROWS_PT, K)
    zq = jnp.zeros((ZQ, 128), jnp.float32)

    h, scr_lo, scr_hi = _encode(
        pos_f, vel_f, W_pos, b_pos.reshape(1, HID),
        W_vel, b_vel.reshape(1, HID), W_node, b_node.reshape(1, HID))
    acc2 = _sc_aggregate(scr_lo, scr_hi, src3, dst3, zq)
    agg_lo = acc2[0].reshape(4 * QROWS, HHALF)[:N]
    agg_hi = acc2[1].reshape(4 * QROWS, HHALF)[:N]
    pos_hat, vel_hat = _process(
        h, agg_lo, agg_hi, W_self, W_nbr, b_proc.reshape(1, HID),
        W_posdec, b_posdec.reshape(1, 2), W_veldec, b_veldec.reshape(1, 2))
    return (pos_hat, vel_hat)
